# baseline (speedup 1.0000x reference)
import functools

import jax
import jax.numpy as jnp
from jax import lax
from jax.experimental import pallas as pl
from jax.experimental.pallas import tpu as pltpu
from jax.experimental.pallas import tpu_sc as plsc

B, N, K, C, H, O = 2, 10000, 16, 128, 128, 128
BN = B * N
NW = 32
BN_PAD = 20480
NODES_PER_TILE = BN_PAD // NW
NODE_CHUNK = 8
CK = NODE_CHUNK * K
CH = NODE_CHUNK * H
CHUNKS = NODES_PER_TILE // NODE_CHUNK
CHUNKS_A = 98
CHUNKS_B = 2 * CHUNKS - CHUNKS_A
PROJ_BLK = 1024
ROW_BLK = 1000
LANES = 16
JV = H // LANES


def _proj_body(f_ref, w1a_ref, w1b_ref, b1_ref, v_ref, e_ref, p_ref, q_ref, vb_ref):
    f = f_ref[...]
    p_ref[...] = (jnp.dot(f, w1a_ref[...], preferred_element_type=jnp.float32)
                  + b1_ref[...]).astype(jnp.bfloat16)
    q_ref[...] = jnp.dot(f, w1b_ref[...],
                         preferred_element_type=jnp.float32).astype(jnp.bfloat16)
    vb_ref[...] = jnp.dot(v_ref[...], e_ref[...],
                          preferred_element_type=jnp.float32).astype(jnp.bfloat16)


def _project(feats2, w1a, w1b, b1row, valid2, expander):
    return pl.pallas_call(
        _proj_body,
        grid=(BN_PAD // PROJ_BLK,),
        in_specs=[
            pl.BlockSpec((PROJ_BLK, C), lambda i: (i, 0)),
            pl.BlockSpec((C, H), lambda i: (0, 0)),
            pl.BlockSpec((C, H), lambda i: (0, 0)),
            pl.BlockSpec((1, H), lambda i: (0, 0)),
            pl.BlockSpec((PROJ_BLK, K), lambda i: (i, 0)),
            pl.BlockSpec((K, K * LANES), lambda i: (0, 0)),
        ],
        out_specs=[
            pl.BlockSpec((PROJ_BLK, H), lambda i: (i, 0)),
            pl.BlockSpec((PROJ_BLK, H), lambda i: (i, 0)),
            pl.BlockSpec((PROJ_BLK, K * LANES), lambda i: (i, 0)),
        ],
        out_shape=[
            jax.ShapeDtypeStruct((BN_PAD, H), jnp.bfloat16),
            jax.ShapeDtypeStruct((BN_PAD, H), jnp.bfloat16),
            jax.ShapeDtypeStruct((BN_PAD, K * LANES), jnp.bfloat16),
        ],
    )(feats2, w1a, w1b, b1row, valid2, expander)


def _sc_body(q_hbm, p_hbm, gidx_hbm, vbc_hbm, hsum_hbm,
             idx0, idx1, rows0, rows1, p0, p1, vv0, vv1, out0, out1,
             sg0, sg1, si0, si1):
    cix = lax.axis_index("c")
    six = lax.axis_index("s")
    my_chunks = jnp.where(cix == 0, CHUNKS_A, CHUNKS_B)
    chunk0 = jnp.where(cix == 0, six * CHUNKS_A,
                       16 * CHUNKS_A + six * CHUNKS_B)
    base = chunk0 * NODE_CHUNK

    idxs = (idx0, idx1)
    rows = (rows0, rows1)
    ps = (p0, p1)
    vvs = (vv0, vv1)
    outs = (out0, out1)
    sgs = (sg0, sg1)
    sis = (si0, si1)

    def in_copy_args(c, s):
        nb = base + c * NODE_CHUNK
        return (
            (gidx_hbm.at[pl.ds(nb * K, CK)], idxs[s], sis[s]),
            (p_hbm.at[pl.ds(nb * (H // 2), CH // 2)], ps[s], sis[s]),
            (vbc_hbm.at[pl.ds(nb * K * 8, CK * 8)], vvs[s], sis[s]),
        )

    def compute_chunk(s, nb):
        rv, pv, vv, ov = rows[s], ps[s], vvs[s], outs[s]

        def node_body(i, c2):
            vks = []
            for t in range(K // 2):
                w = vv[pl.ds(i * K * 8 + t * LANES, LANES)]
                va, vb = plsc.unpack(plsc.bitcast(w, jnp.bfloat16),
                                     format=plsc.PackFormat.INTERLEAVED)
                vks.extend((va, vb))
            for g in range(H // 32):
                pw = pv[pl.ds(i * (H // 2) + g * LANES, LANES)]
                pa, pb = plsc.unpack(plsc.bitcast(pw, jnp.bfloat16),
                                     format=plsc.PackFormat.INTERLEAVED)
                acc_a = jnp.zeros((LANES,), jnp.float32)
                acc_b = jnp.zeros((LANES,), jnp.float32)
                for kk in range(K):
                    qw = rv[i * K + kk, pl.ds(g * LANES, LANES)]
                    qa, qb = plsc.unpack(plsc.bitcast(qw, jnp.bfloat16),
                                         format=plsc.PackFormat.INTERLEAVED)
                    acc_a = acc_a + vks[kk] * jnp.maximum(pa + qa, 0.0)
                    acc_b = acc_b + vks[kk] * jnp.maximum(pb + qb, 0.0)
                ov[pl.ds(i * H + g * 32, LANES)] = acc_a
                ov[pl.ds(i * H + g * 32 + LANES, LANES)] = acc_b
            return c2

        lax.fori_loop(0, NODE_CHUNK, node_body, 0)
        pltpu.sync_copy(ov, hsum_hbm.at[pl.ds(nb * H, CH)])

    def section(m, s):
        @pl.when(m + 1 < my_chunks)
        def _():
            for a in in_copy_args(m + 1, 1 - s):
                pltpu.make_async_copy(*a).wait()
            pltpu.async_copy(q_hbm.at[idxs[1 - s]], rows[1 - s], sgs[1 - s])

        pltpu.make_async_copy(q_hbm.at[idxs[s]], rows[s], sgs[s]).wait()
        compute_chunk(s, base + m * NODE_CHUNK)

        @pl.when(m + 2 < my_chunks)
        def _():
            for a in in_copy_args(m + 2, s):
                pltpu.async_copy(*a)

    for src, dst, _sem in in_copy_args(0, 0):
        pltpu.sync_copy(src, dst)
    pltpu.async_copy(q_hbm.at[idx0], rows0, sg0)
    for a in in_copy_args(1, 1):
        pltpu.async_copy(*a)

    def pair_body(t, carry):
        section(2 * t, 0)
        section(2 * t + 1, 1)
        return carry

    lax.fori_loop(0, my_chunks // 2, pair_body, 0)


def _sc_gather_reduce(q2, p2, gidx, vbc):
    mesh = plsc.VectorSubcoreMesh(core_axis_name="c", subcore_axis_name="s")
    fn = pl.kernel(
        _sc_body,
        out_type=jax.ShapeDtypeStruct((BN_PAD * H,), jnp.float32),
        mesh=mesh,
        compiler_params=pltpu.CompilerParams(needs_layout_passes=False,
                                             use_tc_tiling_on_sc=False),
        scratch_types=[
            pltpu.VMEM((CK,), jnp.int32),
            pltpu.VMEM((CK,), jnp.int32),
            pltpu.VMEM((CK, H // 2), jnp.int32),
            pltpu.VMEM((CK, H // 2), jnp.int32),
            pltpu.VMEM((CH // 2,), jnp.int32),
            pltpu.VMEM((CH // 2,), jnp.int32),
            pltpu.VMEM((CK * 8,), jnp.int32),
            pltpu.VMEM((CK * 8,), jnp.int32),
            pltpu.VMEM((CH,), jnp.float32),
            pltpu.VMEM((CH,), jnp.float32),
            pltpu.SemaphoreType.DMA,
            pltpu.SemaphoreType.DMA,
            pltpu.SemaphoreType.DMA,
            pltpu.SemaphoreType.DMA,
        ],
    )
    return fn(q2, p2, gidx, vbc)


def _out_body(h_ref, v_ref, w2_ref, b2rep_ref, o_ref):
    o_ref[...] = (jnp.dot(h_ref[...], w2_ref[...], preferred_element_type=jnp.float32)
                  + jnp.dot(v_ref[...], b2rep_ref[...], preferred_element_type=jnp.float32))


def _finish(hsum, valid2, w2, b2rep):
    return pl.pallas_call(
        _out_body,
        grid=(BN // ROW_BLK,),
        in_specs=[
            pl.BlockSpec((ROW_BLK, H), lambda i: (i, 0)),
            pl.BlockSpec((ROW_BLK, K), lambda i: (i, 0)),
            pl.BlockSpec((H, O), lambda i: (0, 0)),
            pl.BlockSpec((K, O), lambda i: (0, 0)),
        ],
        out_specs=pl.BlockSpec((ROW_BLK, O), lambda i: (i, 0)),
        out_shape=jax.ShapeDtypeStruct((BN, O), jnp.float32),
    )(hsum, valid2, w2, b2rep)


def kernel(keys, points, feats, n_idxs, neighbor_rel, neighbor_valid, W1, b1, W2, b2):
    feats2 = feats.reshape(BN, C)
    perm = jnp.concatenate(
        [jnp.concatenate([g * 32 + jnp.arange(0, 32, 2),
                          g * 32 + jnp.arange(1, 32, 2)]) for g in range(H // 32)])
    w1a = W1[:C]
    w1b = W1[C:]
    b1row = b1.reshape(1, H)
    valid2 = neighbor_valid.reshape(BN, K)
    pp = jnp.arange(K * LANES)
    expander = ((pp[None, :] // 32 == jnp.arange(K)[:, None] // 2)
                & (pp[None, :] % 2 == jnp.arange(K)[:, None] % 2)).astype(jnp.float32)
    p2, q2, vbc = _project(feats2, w1a, w1b, b1row, valid2, expander)

    q2i = jax.lax.bitcast_convert_type(q2.reshape(BN_PAD, H // 2, 2), jnp.int32)
    p2i = jax.lax.bitcast_convert_type(p2.reshape(BN_PAD, H // 2, 2),
                                       jnp.int32).reshape(BN_PAD * (H // 2))
    vbci = jax.lax.bitcast_convert_type(vbc.reshape(BN_PAD, K * 8, 2),
                                        jnp.int32).reshape(BN_PAD * K * 8)
    gidx = (n_idxs.astype(jnp.int32)
            + (jnp.arange(B, dtype=jnp.int32) * N)[:, None, None]).reshape(BN, K)
    gidx_pad = jnp.zeros((BN_PAD, K), jnp.int32).at[:BN].set(gidx).reshape(BN_PAD * K)
    hsum = _sc_gather_reduce(q2i, p2i, gidx_pad, vbci)
    hsum = hsum[:BN * H].reshape(BN, H)

    b2rep = jnp.broadcast_to(b2[None, :], (K, O))
    out = _finish(hsum, neighbor_valid.reshape(BN, K), W2[perm, :], b2rep)
    return out.reshape(B, N, O)


# speedup vs baseline: 1.4363x; 1.4363x over previous
import functools

import jax
import jax.numpy as jnp
from jax import lax
from jax.experimental import pallas as pl
from jax.experimental.pallas import tpu as pltpu
from jax.experimental.pallas import tpu_sc as plsc

B, N, K, C, H, O = 2, 10000, 16, 128, 128, 128
BN = B * N
NW = 32
BN_PAD = 20480
NODES_PER_TILE = BN_PAD // NW
NODE_CHUNK = 8
CK = NODE_CHUNK * K
CH = NODE_CHUNK * H
CHUNKS = NODES_PER_TILE // NODE_CHUNK
CHUNKS_A = 98
CHUNKS_B = 2 * CHUNKS - CHUNKS_A
PROJ_BLK = 1024
ROW_BLK = 1000
LANES = 16
JV = H // LANES


def _proj_body(f_ref, w1a_ref, w1b_ref, b1_ref, v_ref, e_ref, p_ref, q_ref, vb_ref):
    f = f_ref[...]
    p_ref[...] = jnp.dot(f, w1a_ref[...], preferred_element_type=jnp.float32) + b1_ref[...]
    q_ref[...] = jnp.dot(f, w1b_ref[...],
                         preferred_element_type=jnp.float32).astype(jnp.bfloat16)
    vb_ref[...] = jnp.dot(v_ref[...], e_ref[...], preferred_element_type=jnp.float32)


def _project(feats2, w1a, w1b, b1row, valid2, expander):
    return pl.pallas_call(
        _proj_body,
        grid=(BN_PAD // PROJ_BLK,),
        in_specs=[
            pl.BlockSpec((PROJ_BLK, C), lambda i: (i, 0)),
            pl.BlockSpec((C, H), lambda i: (0, 0)),
            pl.BlockSpec((C, H), lambda i: (0, 0)),
            pl.BlockSpec((1, H), lambda i: (0, 0)),
            pl.BlockSpec((PROJ_BLK, K), lambda i: (i, 0)),
            pl.BlockSpec((K, K * LANES), lambda i: (0, 0)),
        ],
        out_specs=[
            pl.BlockSpec((PROJ_BLK, H), lambda i: (i, 0)),
            pl.BlockSpec((PROJ_BLK, H), lambda i: (i, 0)),
            pl.BlockSpec((PROJ_BLK, K * LANES), lambda i: (i, 0)),
        ],
        out_shape=[
            jax.ShapeDtypeStruct((BN_PAD, H), jnp.float32),
            jax.ShapeDtypeStruct((BN_PAD, H), jnp.bfloat16),
            jax.ShapeDtypeStruct((BN_PAD, K * LANES), jnp.float32),
        ],
    )(feats2, w1a, w1b, b1row, valid2, expander)


def _sc_body(q_hbm, p_hbm, gidx_hbm, vbc_hbm, hsum_hbm,
             idx0, idx1, rows0, rows1, p0, p1, vv0, vv1, out0, out1,
             sg0, sg1, si0, si1):
    cix = lax.axis_index("c")
    six = lax.axis_index("s")
    my_chunks = jnp.where(cix == 0, CHUNKS_A, CHUNKS_B)
    chunk0 = jnp.where(cix == 0, six * CHUNKS_A,
                       16 * CHUNKS_A + six * CHUNKS_B)
    base = chunk0 * NODE_CHUNK

    idxs = (idx0, idx1)
    rows = (rows0, rows1)
    ps = (p0, p1)
    vvs = (vv0, vv1)
    outs = (out0, out1)
    sgs = (sg0, sg1)
    sis = (si0, si1)

    def in_copy_args(c, s):
        nb = base + c * NODE_CHUNK
        return (
            (gidx_hbm.at[pl.ds(nb * K, CK)], idxs[s], sis[s]),
            (p_hbm.at[pl.ds(nb * H, CH)], ps[s], sis[s]),
            (vbc_hbm.at[pl.ds(nb * K * LANES, CK * LANES)], vvs[s], sis[s]),
        )

    def compute_chunk(s, nb):
        rv, pv, vv, ov = rows[s], ps[s], vvs[s], outs[s]

        def node_body(i, c2):
            vks = [vv[pl.ds(i * K * LANES + kk * LANES, LANES)] for kk in range(K)]
            for g in range(H // 32):
                pa = pv[pl.ds(i * H + g * 32, LANES)]
                pb = pv[pl.ds(i * H + g * 32 + LANES, LANES)]
                acc_a = jnp.zeros((LANES,), jnp.float32)
                acc_b = jnp.zeros((LANES,), jnp.float32)
                for kk in range(K):
                    qw = rv[i * K + kk, pl.ds(g * LANES, LANES)]
                    qa, qb = plsc.unpack(plsc.bitcast(qw, jnp.bfloat16),
                                         format=plsc.PackFormat.INTERLEAVED)
                    acc_a = acc_a + vks[kk] * jnp.maximum(pa + qa, 0.0)
                    acc_b = acc_b + vks[kk] * jnp.maximum(pb + qb, 0.0)
                ov[pl.ds(i * H + g * 32, LANES)] = acc_a
                ov[pl.ds(i * H + g * 32 + LANES, LANES)] = acc_b
            return c2

        lax.fori_loop(0, NODE_CHUNK, node_body, 0)
        pltpu.sync_copy(ov, hsum_hbm.at[pl.ds(nb * H, CH)])

    def section(m, s):
        @pl.when(m + 1 < my_chunks)
        def _():
            for a in in_copy_args(m + 1, 1 - s):
                pltpu.make_async_copy(*a).wait()
            pltpu.async_copy(q_hbm.at[idxs[1 - s]], rows[1 - s], sgs[1 - s])

        pltpu.make_async_copy(q_hbm.at[idxs[s]], rows[s], sgs[s]).wait()
        compute_chunk(s, base + m * NODE_CHUNK)

        @pl.when(m + 2 < my_chunks)
        def _():
            for a in in_copy_args(m + 2, s):
                pltpu.async_copy(*a)

    for src, dst, _sem in in_copy_args(0, 0):
        pltpu.sync_copy(src, dst)
    pltpu.async_copy(q_hbm.at[idx0], rows0, sg0)
    for a in in_copy_args(1, 1):
        pltpu.async_copy(*a)

    def pair_body(t, carry):
        section(2 * t, 0)
        section(2 * t + 1, 1)
        return carry

    lax.fori_loop(0, my_chunks // 2, pair_body, 0)


def _sc_gather_reduce(q2, p2, gidx, vbc):
    mesh = plsc.VectorSubcoreMesh(core_axis_name="c", subcore_axis_name="s")
    fn = pl.kernel(
        _sc_body,
        out_type=jax.ShapeDtypeStruct((BN_PAD * H,), jnp.float32),
        mesh=mesh,
        compiler_params=pltpu.CompilerParams(needs_layout_passes=False,
                                             use_tc_tiling_on_sc=False),
        scratch_types=[
            pltpu.VMEM((CK,), jnp.int32),
            pltpu.VMEM((CK,), jnp.int32),
            pltpu.VMEM((CK, H // 2), jnp.int32),
            pltpu.VMEM((CK, H // 2), jnp.int32),
            pltpu.VMEM((CH,), jnp.float32),
            pltpu.VMEM((CH,), jnp.float32),
            pltpu.VMEM((CK * LANES,), jnp.float32),
            pltpu.VMEM((CK * LANES,), jnp.float32),
            pltpu.VMEM((CH,), jnp.float32),
            pltpu.VMEM((CH,), jnp.float32),
            pltpu.SemaphoreType.DMA,
            pltpu.SemaphoreType.DMA,
            pltpu.SemaphoreType.DMA,
            pltpu.SemaphoreType.DMA,
        ],
    )
    return fn(q2, p2.reshape(BN_PAD * H), gidx, vbc.reshape(BN_PAD * K * LANES))


def _out_body(h_ref, v_ref, w2_ref, b2rep_ref, o_ref):
    o_ref[...] = (jnp.dot(h_ref[...], w2_ref[...], preferred_element_type=jnp.float32)
                  + jnp.dot(v_ref[...], b2rep_ref[...], preferred_element_type=jnp.float32))


def _finish(hsum, valid2, w2, b2rep):
    return pl.pallas_call(
        _out_body,
        grid=(BN // ROW_BLK,),
        in_specs=[
            pl.BlockSpec((ROW_BLK, H), lambda i: (i, 0)),
            pl.BlockSpec((ROW_BLK, K), lambda i: (i, 0)),
            pl.BlockSpec((H, O), lambda i: (0, 0)),
            pl.BlockSpec((K, O), lambda i: (0, 0)),
        ],
        out_specs=pl.BlockSpec((ROW_BLK, O), lambda i: (i, 0)),
        out_shape=jax.ShapeDtypeStruct((BN, O), jnp.float32),
    )(hsum, valid2, w2, b2rep)


def kernel(keys, points, feats, n_idxs, neighbor_rel, neighbor_valid, W1, b1, W2, b2):
    feats2 = feats.reshape(BN, C)
    perm = jnp.concatenate(
        [jnp.concatenate([g * 32 + jnp.arange(0, 32, 2),
                          g * 32 + jnp.arange(1, 32, 2)]) for g in range(H // 32)])
    w1a = W1[:C][:, perm]
    w1b = W1[C:]
    b1row = b1[perm].reshape(1, H)
    valid2 = neighbor_valid.reshape(BN, K)
    expander = jnp.repeat(jnp.eye(K, dtype=jnp.float32), LANES, axis=1)
    p2, q2, vbc = _project(feats2, w1a, w1b, b1row, valid2, expander)

    q2i = jax.lax.bitcast_convert_type(q2.reshape(BN_PAD, H // 2, 2), jnp.int32)
    gidx = (n_idxs.astype(jnp.int32)
            + (jnp.arange(B, dtype=jnp.int32) * N)[:, None, None]).reshape(BN, K)
    gidx_pad = jnp.zeros((BN_PAD, K), jnp.int32).at[:BN].set(gidx).reshape(BN_PAD * K)
    hsum = _sc_gather_reduce(q2i, p2, gidx_pad, vbc)
    hsum = hsum[:BN * H].reshape(BN, H)

    b2rep = jnp.broadcast_to(b2[None, :], (K, O))
    out = _finish(hsum, neighbor_valid.reshape(BN, K), W2[perm, :], b2rep)
    return out.reshape(B, N, O)
